# baseline (speedup 1.0000x reference)
import functools

import jax
import jax.numpy as jnp
from jax import lax
from jax.experimental import pallas as pl
from jax.experimental.pallas import tpu as pltpu
from jax.experimental.pallas import tpu_sc as plsc

N = 2560
E = 81920
H = 256
D_IN = 128
D_OUT = 2

NC = 2
NS = 16
LANES = 16

NCHUNK = 8
ROWS = N // NCHUNK
CHUNK_W = ROWS * N
SLICE_W = CHUNK_W // NS
EPT = E // NS
EROWS = EPT // 128

def _dot(a, b):
    ah = a.astype(jnp.bfloat16)
    al = (a - ah.astype(jnp.float32)).astype(jnp.bfloat16)
    bh = b.astype(jnp.bfloat16)
    bl = (b - bh.astype(jnp.float32)).astype(jnp.bfloat16)
    dims = (((1,), (0,)), ((), ()))

    def d(u, v):
        return jax.lax.dot_general(u, v, dims,
                                   preferred_element_type=jnp.float32)

    return d(ah, bh) + (d(ah, bl) + d(al, bh))


def _dot1(a, b):
    return jax.lax.dot_general(a.astype(jnp.bfloat16), b.astype(jnp.bfloat16),
                               (((1,), (0,)), ((), ())),
                               preferred_element_type=jnp.float32)


def _lrelu(v):
    return jnp.maximum(v, 0.2 * v)


def _pre1_body(x_ref, w_ref, as_ref, ad_ref,
               xl_ref, asrc_ref, adst_ref, amax_ref, exs_ref):
    xl = _dot1(x_ref[...], w_ref[...])
    xl_ref[...] = xl
    a_s = _dot(xl, as_ref[...])
    a_d = _dot(xl, ad_ref[...])
    asrc_ref[...] = a_s
    adst_ref[...] = a_d
    amax = jnp.max(a_s)
    amax_ref[...] = jnp.full((1, 1), amax, jnp.float32)
    m = _lrelu(amax + a_d)
    exs_ref[...] = jnp.exp(_lrelu(a_s + a_d) - m)


def _pre1(x, w, att_s, att_d):
    return pl.pallas_call(
        _pre1_body,
        out_shape=[
            jax.ShapeDtypeStruct((N, H), jnp.float32),
            jax.ShapeDtypeStruct((N, 1), jnp.float32),
            jax.ShapeDtypeStruct((N, 1), jnp.float32),
            jax.ShapeDtypeStruct((1, 1), jnp.float32),
            jax.ShapeDtypeStruct((N, 1), jnp.float32),
        ],
    )(x, w, att_s, att_d)


def _pre2_body(raw_ref, g_ref, bt_ref, w_ref, as_ref, ad_ref,
               xl_ref, asrc_ref, adst_ref, amax_ref, exs_ref):
    r = raw_ref[...]
    mean = jnp.mean(r, axis=0, keepdims=True)
    d = r - mean
    var = jnp.mean(d * d, axis=0, keepdims=True)
    h = jnp.maximum(d * jax.lax.rsqrt(var + 1e-5) * g_ref[...] + bt_ref[...],
                    0.0)
    xl = _dot1(h, w_ref[...])
    xl_ref[...] = xl
    a_s = _dot(xl, as_ref[...])
    a_d = _dot(xl, ad_ref[...])
    asrc_ref[...] = a_s
    adst_ref[...] = a_d
    amax = jnp.max(a_s)
    amax_ref[...] = jnp.full((1, 1), amax, jnp.float32)
    m = _lrelu(amax + a_d)
    exs_ref[...] = jnp.exp(_lrelu(a_s + a_d) - m)


def _pre2(raw, gamma, beta, w, att_s, att_d):
    return pl.pallas_call(
        _pre2_body,
        out_shape=[
            jax.ShapeDtypeStruct((N, H), jnp.float32),
            jax.ShapeDtypeStruct((N, 1), jnp.float32),
            jax.ShapeDtypeStruct((N, 1), jnp.float32),
            jax.ShapeDtypeStruct((1, 1), jnp.float32),
            jax.ShapeDtypeStruct((N, 1), jnp.float32),
        ],
    )(raw, gamma, beta, w, att_s, att_d)


def _sc_scatter_body(src_hbm, dst_hbm, asrc_hbm, adst_hbm, amax_hbm, zeros_hbm,
                     p_hbm,
                     amax_v, src_v, dst_v, ag_v, dg_v, ex_v, val_v, idx_v,
                     dmy_val, dmy_idx, p_sh, sem):
    c = lax.axis_index("c")
    s = lax.axis_index("s")
    pltpu.sync_copy(amax_hbm, amax_v)
    pltpu.sync_copy(src_hbm.at[s], src_v)
    pltpu.sync_copy(dst_hbm.at[s], dst_v)
    amx = amax_v[...]

    cp1 = pltpu.async_copy(asrc_hbm.at[src_v], ag_v, sem)
    cp2 = pltpu.async_copy(adst_hbm.at[dst_v], dg_v, sem)
    cp1.wait()
    cp2.wait()

    def ex_row(r, carry):
        sl = pl.ds(r * LANES, LANES)
        ag = ag_v[sl]
        dg = dg_v[sl]
        t = ag + dg
        alpha = jnp.maximum(t, 0.2 * t)
        m0 = amx + dg
        m = jnp.maximum(m0, 0.2 * m0)
        ex_v[sl] = jnp.exp(alpha - m)
        return carry

    lax.fori_loop(0, EPT // LANES, ex_row, 0)

    def zero_dummy(r, carry):
        sl = pl.ds(r * LANES, LANES)
        dmy_val[sl] = jnp.zeros((LANES,), jnp.float32)
        dmy_idx[sl] = jnp.zeros((LANES,), jnp.int32)
        return carry

    lax.fori_loop(0, 128 // LANES, zero_dummy, 0)

    for k in range(NCHUNK // NC):
        cid = c * (NCHUNK // NC) + k
        lo = cid * ROWS
        pltpu.sync_copy(zeros_hbm.at[pl.ds(s * SLICE_W, SLICE_W)],
                        p_sh.at[pl.ds(s * SLICE_W, SLICE_W)])
        plsc.subcore_barrier()
        kbase = k * EPT

        def compute_row(r, carry):
            sl = pl.ds(kbase + r * LANES, LANES)
            sle = pl.ds(r * LANES, LANES)
            sv = src_v[sle]
            dv = dst_v[sle]
            ex = ex_v[sle]
            dloc = dv - lo
            inrng = (dloc >= 0) & (dloc < ROWS)
            val_v[sl] = jnp.where(inrng, ex, 0.0)
            idx_v[sl] = jnp.where(inrng, dloc * N + sv, 0)
            return carry

        lax.fori_loop(0, EPT // LANES, compute_row, 0)

        pltpu.sync_copy(val_v.at[pl.ds(kbase, EPT)],
                        p_sh.at[idx_v.at[pl.ds(kbase, EPT)]], add=True)
        pltpu.sync_copy(dmy_val, p_sh.at[dmy_idx], add=True)
        pltpu.sync_copy(dmy_val, p_sh.at[dmy_idx], add=True)
        plsc.subcore_barrier()
        pltpu.sync_copy(p_sh.at[pl.ds(s * SLICE_W, SLICE_W)],
                        p_hbm.at[pl.ds(cid * CHUNK_W + s * SLICE_W, SLICE_W)])
        if k < NCHUNK // NC - 1:
            plsc.subcore_barrier()


def _sc_scatter(src, dst, asrc, adst, amax16, zeros):
    mesh = plsc.VectorSubcoreMesh(core_axis_name="c", subcore_axis_name="s")
    f = pl.kernel(
        _sc_scatter_body,
        out_type=jax.ShapeDtypeStruct((N * N,), jnp.float32),
        mesh=mesh,
        scratch_types=[
            pltpu.VMEM((LANES,), jnp.float32),
            pltpu.VMEM((EPT,), jnp.int32),
            pltpu.VMEM((EPT,), jnp.int32),
            pltpu.VMEM((EPT,), jnp.float32),
            pltpu.VMEM((EPT,), jnp.float32),
            pltpu.VMEM((EPT,), jnp.float32),
            pltpu.VMEM(((NCHUNK // NC) * EPT,), jnp.float32),
            pltpu.VMEM(((NCHUNK // NC) * EPT,), jnp.int32),
            pltpu.VMEM((128,), jnp.float32),
            pltpu.VMEM((128,), jnp.int32),
            pltpu.VMEM_SHARED((CHUNK_W,), jnp.float32),
            pltpu.SemaphoreType.DMA,
        ],
    )
    return f(src, dst, asrc, adst, amax16, zeros)


_MID_BLK = 320


def _mid_body(p_ref, xl_ref, xlr_ref, exs_ref, b_ref, raw_ref):
    p = p_ref[...]
    acc = _dot(p, xl_ref[...])
    denom = jnp.sum(p, axis=1, keepdims=True) + exs_ref[...] + 1e-16
    raw_ref[...] = (acc + exs_ref[...] * xlr_ref[...]) / denom + b_ref[...]


def _mid(p, xl, exs, b):
    g = N // _MID_BLK
    return pl.pallas_call(
        _mid_body,
        grid=(g,),
        in_specs=[
            pl.BlockSpec((_MID_BLK, N), lambda i: (i, 0)),
            pl.BlockSpec((N, H), lambda i: (0, 0)),
            pl.BlockSpec((_MID_BLK, H), lambda i: (i, 0)),
            pl.BlockSpec((_MID_BLK, 1), lambda i: (i, 0)),
            pl.BlockSpec((1, H), lambda i: (0, 0)),
        ],
        out_specs=pl.BlockSpec((_MID_BLK, H), lambda i: (i, 0)),
        out_shape=jax.ShapeDtypeStruct((N, H), jnp.float32),
    )(p, xl, xl, exs, b)


def _post_body(raw_ref, g_ref, bt_ref, wfc_ref, bfc_ref, mask_ref, out_ref):
    r = raw_ref[...]
    mean = jnp.mean(r, axis=0, keepdims=True)
    d = r - mean
    var = jnp.mean(d * d, axis=0, keepdims=True)
    h = jnp.maximum(d * jax.lax.rsqrt(var + 1e-5) * g_ref[...] + bt_ref[...],
                    0.0)
    o = _dot1(h, wfc_ref[...]) + bfc_ref[...]
    out_ref[...] = o * mask_ref[...]


def _post(raw, gamma, beta, wfc, bfc, mask):
    return pl.pallas_call(
        _post_body,
        out_shape=jax.ShapeDtypeStruct((N, D_OUT), jnp.float32),
    )(raw, gamma, beta, wfc, bfc, mask)


def kernel(x, edge_index, mask, W1, att_src1, att_dst1, b1, gamma1, beta1,
           W2, att_src2, att_dst2, b2, gamma2, beta2, Wfc, bfc):
    src = edge_index[0].reshape(NS, EPT)
    dst = edge_index[1].reshape(NS, EPT)
    zeros = jnp.zeros((CHUNK_W,), jnp.float32)

    xl1, asrc1, adst1, amax1, exs1 = _pre1(
        x, W1, att_src1.reshape(H, 1), att_dst1.reshape(H, 1))
    p1 = _sc_scatter(src, dst, asrc1.reshape(N), adst1.reshape(N),
                     jnp.broadcast_to(amax1.reshape(1), (LANES,)), zeros)
    raw1 = _mid(p1.reshape(N, N), xl1, exs1, b1.reshape(1, H))

    xl2, asrc2, adst2, amax2, exs2 = _pre2(
        raw1, gamma1.reshape(1, H), beta1.reshape(1, H), W2,
        att_src2.reshape(H, 1), att_dst2.reshape(H, 1))
    p2 = _sc_scatter(src, dst, asrc2.reshape(N), adst2.reshape(N),
                     jnp.broadcast_to(amax2.reshape(1), (LANES,)), zeros)
    raw2 = _mid(p2.reshape(N, N), xl2, exs2, b2.reshape(1, H))

    out = _post(raw2, gamma2.reshape(1, H), beta2.reshape(1, H),
                Wfc, bfc.reshape(1, D_OUT), mask.reshape(N, 1))
    return out.reshape(64, 40, 2)


# speedup vs baseline: 2.4589x; 2.4589x over previous
import functools

import jax
import jax.numpy as jnp
from jax import lax
from jax.experimental import pallas as pl
from jax.experimental.pallas import tpu as pltpu
from jax.experimental.pallas import tpu_sc as plsc

N = 2560
E = 81920
H = 256
D_IN = 128
D_OUT = 2

NC = 2
NS = 16
LANES = 16

NCHUNK = 8
ROWS = N // NCHUNK
CHUNK_W = ROWS * N
SLICE_W = CHUNK_W // NS
EPT = E // NS
EROWS = EPT // 128

def _dot(a, b):
    ah = a.astype(jnp.bfloat16)
    al = (a - ah.astype(jnp.float32)).astype(jnp.bfloat16)
    bh = b.astype(jnp.bfloat16)
    bl = (b - bh.astype(jnp.float32)).astype(jnp.bfloat16)
    dims = (((1,), (0,)), ((), ()))

    def d(u, v):
        return jax.lax.dot_general(u, v, dims,
                                   preferred_element_type=jnp.float32)

    return d(ah, bh) + (d(ah, bl) + d(al, bh))


def _dot1(a, b):
    return jax.lax.dot_general(a.astype(jnp.bfloat16), b.astype(jnp.bfloat16),
                               (((1,), (0,)), ((), ())),
                               preferred_element_type=jnp.float32)


def _lrelu(v):
    return jnp.maximum(v, 0.2 * v)


def _pre1_body(x_ref, w_ref, as_ref, ad_ref,
               xl_ref, asrc_ref, adst_ref, amax_ref, exs_ref):
    xl = _dot1(x_ref[...], w_ref[...])
    xl_ref[...] = xl
    a_s = _dot(xl, as_ref[...])
    a_d = _dot(xl, ad_ref[...])
    asrc_ref[...] = a_s
    adst_ref[...] = a_d
    amax = jnp.max(a_s)
    amax_ref[...] = jnp.full((1, 1), amax, jnp.float32)
    m = _lrelu(amax + a_d)
    exs_ref[...] = jnp.exp(_lrelu(a_s + a_d) - m)


def _pre1(x, w, att_s, att_d):
    return pl.pallas_call(
        _pre1_body,
        out_shape=[
            jax.ShapeDtypeStruct((N, H), jnp.float32),
            jax.ShapeDtypeStruct((N, 1), jnp.float32),
            jax.ShapeDtypeStruct((N, 1), jnp.float32),
            jax.ShapeDtypeStruct((1, 1), jnp.float32),
            jax.ShapeDtypeStruct((N, 1), jnp.float32),
        ],
    )(x, w, att_s, att_d)


def _pre2_body(raw_ref, g_ref, bt_ref, w_ref, as_ref, ad_ref,
               xl_ref, asrc_ref, adst_ref, amax_ref, exs_ref):
    r = raw_ref[...]
    mean = jnp.mean(r, axis=0, keepdims=True)
    d = r - mean
    var = jnp.mean(d * d, axis=0, keepdims=True)
    h = jnp.maximum(d * jax.lax.rsqrt(var + 1e-5) * g_ref[...] + bt_ref[...],
                    0.0)
    xl = _dot1(h, w_ref[...])
    xl_ref[...] = xl
    a_s = _dot(xl, as_ref[...])
    a_d = _dot(xl, ad_ref[...])
    asrc_ref[...] = a_s
    adst_ref[...] = a_d
    amax = jnp.max(a_s)
    amax_ref[...] = jnp.full((1, 1), amax, jnp.float32)
    m = _lrelu(amax + a_d)
    exs_ref[...] = jnp.exp(_lrelu(a_s + a_d) - m)


def _pre2(raw, gamma, beta, w, att_s, att_d):
    return pl.pallas_call(
        _pre2_body,
        out_shape=[
            jax.ShapeDtypeStruct((N, H), jnp.float32),
            jax.ShapeDtypeStruct((N, 1), jnp.float32),
            jax.ShapeDtypeStruct((N, 1), jnp.float32),
            jax.ShapeDtypeStruct((1, 1), jnp.float32),
            jax.ShapeDtypeStruct((N, 1), jnp.float32),
        ],
    )(raw, gamma, beta, w, att_s, att_d)


def _sc_scatter_body(src_hbm, dst_hbm, asrc_hbm, adst_hbm, amax_hbm, zeros_hbm,
                     p_hbm,
                     amax_v, src_v, dst_v, ag_v, dg_v, ex_v, val_v, idx_v,
                     dmy_val, dmy_idx, p_sh, sem):
    c = lax.axis_index("c")
    s = lax.axis_index("s")
    pltpu.sync_copy(amax_hbm, amax_v)
    pltpu.sync_copy(src_hbm.at[s], src_v)
    pltpu.sync_copy(dst_hbm.at[s], dst_v)
    amx = amax_v[...]

    cp1 = pltpu.async_copy(asrc_hbm.at[src_v], ag_v, sem)
    cp2 = pltpu.async_copy(adst_hbm.at[dst_v], dg_v, sem)
    cp1.wait()
    cp2.wait()

    def ex_row(r, carry):
        sl = pl.ds(r * LANES, LANES)
        ag = ag_v[sl]
        dg = dg_v[sl]
        t = ag + dg
        alpha = jnp.maximum(t, 0.2 * t)
        m0 = amx + dg
        m = jnp.maximum(m0, 0.2 * m0)
        ex_v[sl] = jnp.exp(alpha - m)
        return carry

    lax.fori_loop(0, EPT // LANES, ex_row, 0)

    def zero_dummy(r, carry):
        sl = pl.ds(r * LANES, LANES)
        dmy_val[sl] = jnp.zeros((LANES,), jnp.float32)
        dmy_idx[sl] = jnp.zeros((LANES,), jnp.int32)
        return carry

    lax.fori_loop(0, 128 // LANES, zero_dummy, 0)

    for k in range(NCHUNK // NC):
        cid = c * (NCHUNK // NC) + k
        lo = cid * ROWS
        pltpu.sync_copy(zeros_hbm.at[pl.ds(s * SLICE_W, SLICE_W)],
                        p_sh.at[pl.ds(s * SLICE_W, SLICE_W)])
        plsc.subcore_barrier()
        kbase = k * EPT

        def compute_row(r, carry):
            sl = pl.ds(kbase + r * LANES, LANES)
            sle = pl.ds(r * LANES, LANES)
            sv = src_v[sle]
            dv = dst_v[sle]
            ex = ex_v[sle]
            dloc = dv - lo
            inrng = (dloc >= 0) & (dloc < ROWS)
            spread = s * EPT + r * LANES + lax.iota(jnp.int32, LANES)
            val_v[sl] = jnp.where(inrng, ex, 0.0)
            idx_v[sl] = jnp.where(inrng, dloc * N + sv, spread)
            return carry

        lax.fori_loop(0, EPT // LANES, compute_row, 0)

        pltpu.sync_copy(val_v.at[pl.ds(kbase, EPT)],
                        p_sh.at[idx_v.at[pl.ds(kbase, EPT)]], add=True)
        pltpu.sync_copy(dmy_val, p_sh.at[dmy_idx], add=True)
        pltpu.sync_copy(dmy_val, p_sh.at[dmy_idx], add=True)
        plsc.subcore_barrier()
        pltpu.sync_copy(p_sh.at[pl.ds(s * SLICE_W, SLICE_W)],
                        p_hbm.at[pl.ds(cid * CHUNK_W + s * SLICE_W, SLICE_W)])
        if k < NCHUNK // NC - 1:
            plsc.subcore_barrier()


def _sc_scatter(src, dst, asrc, adst, amax16, zeros):
    mesh = plsc.VectorSubcoreMesh(core_axis_name="c", subcore_axis_name="s")
    f = pl.kernel(
        _sc_scatter_body,
        out_type=jax.ShapeDtypeStruct((N * N,), jnp.float32),
        mesh=mesh,
        scratch_types=[
            pltpu.VMEM((LANES,), jnp.float32),
            pltpu.VMEM((EPT,), jnp.int32),
            pltpu.VMEM((EPT,), jnp.int32),
            pltpu.VMEM((EPT,), jnp.float32),
            pltpu.VMEM((EPT,), jnp.float32),
            pltpu.VMEM((EPT,), jnp.float32),
            pltpu.VMEM(((NCHUNK // NC) * EPT,), jnp.float32),
            pltpu.VMEM(((NCHUNK // NC) * EPT,), jnp.int32),
            pltpu.VMEM((128,), jnp.float32),
            pltpu.VMEM((128,), jnp.int32),
            pltpu.VMEM_SHARED((CHUNK_W,), jnp.float32),
            pltpu.SemaphoreType.DMA,
        ],
    )
    return f(src, dst, asrc, adst, amax16, zeros)


_MID_BLK = 320


def _mid_body(p_ref, xl_ref, xlr_ref, exs_ref, b_ref, raw_ref):
    p = p_ref[...]
    acc = _dot(p, xl_ref[...])
    denom = jnp.sum(p, axis=1, keepdims=True) + exs_ref[...] + 1e-16
    raw_ref[...] = (acc + exs_ref[...] * xlr_ref[...]) / denom + b_ref[...]


def _mid(p, xl, exs, b):
    g = N // _MID_BLK
    return pl.pallas_call(
        _mid_body,
        grid=(g,),
        in_specs=[
            pl.BlockSpec((_MID_BLK, N), lambda i: (i, 0)),
            pl.BlockSpec((N, H), lambda i: (0, 0)),
            pl.BlockSpec((_MID_BLK, H), lambda i: (i, 0)),
            pl.BlockSpec((_MID_BLK, 1), lambda i: (i, 0)),
            pl.BlockSpec((1, H), lambda i: (0, 0)),
        ],
        out_specs=pl.BlockSpec((_MID_BLK, H), lambda i: (i, 0)),
        out_shape=jax.ShapeDtypeStruct((N, H), jnp.float32),
    )(p, xl, xl, exs, b)


def _post_body(raw_ref, g_ref, bt_ref, wfc_ref, bfc_ref, mask_ref, out_ref):
    r = raw_ref[...]
    mean = jnp.mean(r, axis=0, keepdims=True)
    d = r - mean
    var = jnp.mean(d * d, axis=0, keepdims=True)
    h = jnp.maximum(d * jax.lax.rsqrt(var + 1e-5) * g_ref[...] + bt_ref[...],
                    0.0)
    o = _dot1(h, wfc_ref[...]) + bfc_ref[...]
    out_ref[...] = o * mask_ref[...]


def _post(raw, gamma, beta, wfc, bfc, mask):
    return pl.pallas_call(
        _post_body,
        out_shape=jax.ShapeDtypeStruct((N, D_OUT), jnp.float32),
    )(raw, gamma, beta, wfc, bfc, mask)


def kernel(x, edge_index, mask, W1, att_src1, att_dst1, b1, gamma1, beta1,
           W2, att_src2, att_dst2, b2, gamma2, beta2, Wfc, bfc):
    src = edge_index[0].reshape(NS, EPT)
    dst = edge_index[1].reshape(NS, EPT)
    zeros = jnp.zeros((CHUNK_W,), jnp.float32)

    xl1, asrc1, adst1, amax1, exs1 = _pre1(
        x, W1, att_src1.reshape(H, 1), att_dst1.reshape(H, 1))
    p1 = _sc_scatter(src, dst, asrc1.reshape(N), adst1.reshape(N),
                     jnp.broadcast_to(amax1.reshape(1), (LANES,)), zeros)
    raw1 = _mid(p1.reshape(N, N), xl1, exs1, b1.reshape(1, H))

    xl2, asrc2, adst2, amax2, exs2 = _pre2(
        raw1, gamma1.reshape(1, H), beta1.reshape(1, H), W2,
        att_src2.reshape(H, 1), att_dst2.reshape(H, 1))
    p2 = _sc_scatter(src, dst, asrc2.reshape(N), adst2.reshape(N),
                     jnp.broadcast_to(amax2.reshape(1), (LANES,)), zeros)
    raw2 = _mid(p2.reshape(N, N), xl2, exs2, b2.reshape(1, H))

    out = _post(raw2, gamma2.reshape(1, H), beta2.reshape(1, H),
                Wfc, bfc.reshape(1, D_OUT), mask.reshape(N, 1))
    return out.reshape(64, 40, 2)


# speedup vs baseline: 2.6355x; 1.0718x over previous
import functools

import jax
import jax.numpy as jnp
from jax import lax
from jax.experimental import pallas as pl
from jax.experimental.pallas import tpu as pltpu
from jax.experimental.pallas import tpu_sc as plsc

N = 2560
E = 81920
H = 256
D_IN = 128
D_OUT = 2

NC = 2
NS = 16
LANES = 16

NCHUNK = 8
ROWS = N // NCHUNK
CHUNK_W = ROWS * N
SLICE_W = CHUNK_W // NS
EPT = E // NS
EROWS = EPT // 128

def _dot(a, b):
    ah = a.astype(jnp.bfloat16)
    al = (a - ah.astype(jnp.float32)).astype(jnp.bfloat16)
    bh = b.astype(jnp.bfloat16)
    bl = (b - bh.astype(jnp.float32)).astype(jnp.bfloat16)
    dims = (((1,), (0,)), ((), ()))

    def d(u, v):
        return jax.lax.dot_general(u, v, dims,
                                   preferred_element_type=jnp.float32)

    return d(ah, bh) + (d(ah, bl) + d(al, bh))


def _dot1(a, b):
    return jax.lax.dot_general(a.astype(jnp.bfloat16), b.astype(jnp.bfloat16),
                               (((1,), (0,)), ((), ())),
                               preferred_element_type=jnp.float32)


def _lrelu(v):
    return jnp.maximum(v, 0.2 * v)


def _pre1_body(x_ref, w_ref, as_ref, ad_ref,
               xl_ref, asrc_ref, adst_ref, amax_ref, exs_ref):
    xl = _dot1(x_ref[...], w_ref[...])
    xl_ref[...] = xl
    a_s = _dot(xl, as_ref[...])
    a_d = _dot(xl, ad_ref[...])
    asrc_ref[...] = a_s
    adst_ref[...] = a_d
    amax = jnp.max(a_s)
    amax_ref[...] = jnp.full((1, 1), amax, jnp.float32)
    m = _lrelu(amax + a_d)
    exs_ref[...] = jnp.exp(_lrelu(a_s + a_d) - m)


def _pre1(x, w, att_s, att_d):
    return pl.pallas_call(
        _pre1_body,
        out_shape=[
            jax.ShapeDtypeStruct((N, H), jnp.float32),
            jax.ShapeDtypeStruct((N, 1), jnp.float32),
            jax.ShapeDtypeStruct((N, 1), jnp.float32),
            jax.ShapeDtypeStruct((1, 1), jnp.float32),
            jax.ShapeDtypeStruct((N, 1), jnp.float32),
        ],
    )(x, w, att_s, att_d)


def _sc_scatter_body(src_hbm, dst_hbm, asrc_hbm, adst_hbm, amax_hbm, zeros_hbm,
                     p_hbm,
                     amax_v, src_v, dst_v, ag_v, dg_v, ex_v, val_v, idx_v,
                     dmy_val, dmy_idx, p_sh, sem):
    c = lax.axis_index("c")
    s = lax.axis_index("s")
    pltpu.sync_copy(amax_hbm, amax_v)
    pltpu.sync_copy(src_hbm.at[s], src_v)
    pltpu.sync_copy(dst_hbm.at[s], dst_v)
    amx = amax_v[...]

    cp1 = pltpu.async_copy(asrc_hbm.at[src_v], ag_v, sem)
    cp2 = pltpu.async_copy(adst_hbm.at[dst_v], dg_v, sem)
    cp1.wait()
    cp2.wait()

    def ex_row(r, carry):
        sl = pl.ds(r * LANES, LANES)
        ag = ag_v[sl]
        dg = dg_v[sl]
        t = ag + dg
        alpha = jnp.maximum(t, 0.2 * t)
        m0 = amx + dg
        m = jnp.maximum(m0, 0.2 * m0)
        ex_v[sl] = jnp.exp(alpha - m)
        return carry

    lax.fori_loop(0, EPT // LANES, ex_row, 0)

    def zero_dummy(r, carry):
        sl = pl.ds(r * LANES, LANES)
        dmy_val[sl] = jnp.zeros((LANES,), jnp.float32)
        dmy_idx[sl] = jnp.zeros((LANES,), jnp.int32)
        return carry

    lax.fori_loop(0, 128 // LANES, zero_dummy, 0)

    for k in range(NCHUNK // NC):
        cid = c * (NCHUNK // NC) + k
        lo = cid * ROWS
        zcp = pltpu.async_copy(zeros_hbm.at[pl.ds(s * SLICE_W, SLICE_W)],
                               p_sh.at[pl.ds(s * SLICE_W, SLICE_W)], sem)
        kbase = k * EPT

        def compute_row(r, carry):
            sl = pl.ds(kbase + r * LANES, LANES)
            sle = pl.ds(r * LANES, LANES)
            sv = src_v[sle]
            dv = dst_v[sle]
            ex = ex_v[sle]
            dloc = dv - lo
            inrng = (dloc >= 0) & (dloc < ROWS)
            spread = s * EPT + r * LANES + lax.iota(jnp.int32, LANES)
            val_v[sl] = jnp.where(inrng, ex, 0.0)
            idx_v[sl] = jnp.where(inrng, dloc * N + sv, spread)
            return carry

        lax.fori_loop(0, EPT // LANES, compute_row, 0)
        zcp.wait()
        plsc.subcore_barrier()

        pltpu.sync_copy(val_v.at[pl.ds(kbase, EPT)],
                        p_sh.at[idx_v.at[pl.ds(kbase, EPT)]], add=True)
        pltpu.sync_copy(dmy_val, p_sh.at[dmy_idx], add=True)
        pltpu.sync_copy(dmy_val, p_sh.at[dmy_idx], add=True)
        plsc.subcore_barrier()
        pltpu.sync_copy(p_sh.at[pl.ds(s * SLICE_W, SLICE_W)],
                        p_hbm.at[pl.ds(cid * CHUNK_W + s * SLICE_W, SLICE_W)])
        if k < NCHUNK // NC - 1:
            plsc.subcore_barrier()


def _sc_scatter(src, dst, asrc, adst, amax16, zeros):
    mesh = plsc.VectorSubcoreMesh(core_axis_name="c", subcore_axis_name="s")
    f = pl.kernel(
        _sc_scatter_body,
        out_type=jax.ShapeDtypeStruct((N * N,), jnp.float32),
        mesh=mesh,
        scratch_types=[
            pltpu.VMEM((LANES,), jnp.float32),
            pltpu.VMEM((EPT,), jnp.int32),
            pltpu.VMEM((EPT,), jnp.int32),
            pltpu.VMEM((EPT,), jnp.float32),
            pltpu.VMEM((EPT,), jnp.float32),
            pltpu.VMEM((EPT,), jnp.float32),
            pltpu.VMEM(((NCHUNK // NC) * EPT,), jnp.float32),
            pltpu.VMEM(((NCHUNK // NC) * EPT,), jnp.int32),
            pltpu.VMEM((128,), jnp.float32),
            pltpu.VMEM((128,), jnp.int32),
            pltpu.VMEM_SHARED((CHUNK_W,), jnp.float32),
            pltpu.SemaphoreType.DMA,
        ],
    )
    return f(src, dst, asrc, adst, amax16, zeros)


_MID_BLK = 320
_MID_G = N // _MID_BLK


def _mid_block(p_ref, xl_ref, exs_ref, b_ref, raw_s):
    i = pl.program_id(0)
    p = p_ref[...]
    acc = _dot(p, xl_ref[...])
    rows = pl.ds(i * _MID_BLK, _MID_BLK)
    exs = exs_ref[rows, :]
    denom = jnp.sum(p, axis=1, keepdims=True) + exs + 1e-16
    raw_s[rows, :] = (acc + exs * xl_ref[rows, :]) / denom + b_ref[...]


def _bn_relu(r, g, bt):
    mean = jnp.mean(r, axis=0, keepdims=True)
    d = r - mean
    var = jnp.mean(d * d, axis=0, keepdims=True)
    return jnp.maximum(d * jax.lax.rsqrt(var + 1e-5) * g + bt, 0.0)


def _midpre_body(p_ref, xl_ref, exs_ref, b_ref, g_ref, bt_ref, w_ref,
                 as_ref, ad_ref,
                 xl2_ref, asrc_ref, adst_ref, amax_ref, exs2_ref, raw_s):
    _mid_block(p_ref, xl_ref, exs_ref, b_ref, raw_s)

    @pl.when(pl.program_id(0) == _MID_G - 1)
    def _():
        h = _bn_relu(raw_s[...], g_ref[...], bt_ref[...])
        xl = _dot1(h, w_ref[...])
        xl2_ref[...] = xl
        a_s = _dot(xl, as_ref[...])
        a_d = _dot(xl, ad_ref[...])
        asrc_ref[...] = a_s
        adst_ref[...] = a_d
        amax = jnp.max(a_s)
        amax_ref[...] = jnp.full((1, 1), amax, jnp.float32)
        m = _lrelu(amax + a_d)
        exs2_ref[...] = jnp.exp(_lrelu(a_s + a_d) - m)


def _midpre(p, xl, exs, b, gamma, beta, w, att_s, att_d):
    full = pl.BlockSpec(index_map=lambda i: (0, 0))
    return pl.pallas_call(
        _midpre_body,
        grid=(_MID_G,),
        in_specs=[
            pl.BlockSpec((_MID_BLK, N), lambda i: (i, 0)),
            pl.BlockSpec((N, H), lambda i: (0, 0)),
            pl.BlockSpec((N, 1), lambda i: (0, 0)),
            pl.BlockSpec((1, H), lambda i: (0, 0)),
            pl.BlockSpec((1, H), lambda i: (0, 0)),
            pl.BlockSpec((1, H), lambda i: (0, 0)),
            pl.BlockSpec((H, H), lambda i: (0, 0)),
            pl.BlockSpec((H, 1), lambda i: (0, 0)),
            pl.BlockSpec((H, 1), lambda i: (0, 0)),
        ],
        out_specs=[
            pl.BlockSpec((N, H), lambda i: (0, 0)),
            pl.BlockSpec((N, 1), lambda i: (0, 0)),
            pl.BlockSpec((N, 1), lambda i: (0, 0)),
            pl.BlockSpec((1, 1), lambda i: (0, 0)),
            pl.BlockSpec((N, 1), lambda i: (0, 0)),
        ],
        out_shape=[
            jax.ShapeDtypeStruct((N, H), jnp.float32),
            jax.ShapeDtypeStruct((N, 1), jnp.float32),
            jax.ShapeDtypeStruct((N, 1), jnp.float32),
            jax.ShapeDtypeStruct((1, 1), jnp.float32),
            jax.ShapeDtypeStruct((N, 1), jnp.float32),
        ],
        scratch_shapes=[pltpu.VMEM((N, H), jnp.float32)],
    )(p, xl, exs, b, gamma, beta, w, att_s, att_d)


def _midpost_body(p_ref, xl_ref, exs_ref, b_ref, g_ref, bt_ref, wfc_ref,
                  bfc_ref, mask_ref, out_ref, raw_s):
    _mid_block(p_ref, xl_ref, exs_ref, b_ref, raw_s)

    @pl.when(pl.program_id(0) == _MID_G - 1)
    def _():
        h = _bn_relu(raw_s[...], g_ref[...], bt_ref[...])
        o = _dot1(h, wfc_ref[...]) + bfc_ref[...]
        out_ref[...] = o * mask_ref[...]


def _midpost(p, xl, exs, b, gamma, beta, wfc, bfc, mask):
    return pl.pallas_call(
        _midpost_body,
        grid=(_MID_G,),
        in_specs=[
            pl.BlockSpec((_MID_BLK, N), lambda i: (i, 0)),
            pl.BlockSpec((N, H), lambda i: (0, 0)),
            pl.BlockSpec((N, 1), lambda i: (0, 0)),
            pl.BlockSpec((1, H), lambda i: (0, 0)),
            pl.BlockSpec((1, H), lambda i: (0, 0)),
            pl.BlockSpec((1, H), lambda i: (0, 0)),
            pl.BlockSpec((H, D_OUT), lambda i: (0, 0)),
            pl.BlockSpec((1, D_OUT), lambda i: (0, 0)),
            pl.BlockSpec((N, 1), lambda i: (0, 0)),
        ],
        out_specs=pl.BlockSpec((N, D_OUT), lambda i: (0, 0)),
        out_shape=jax.ShapeDtypeStruct((N, D_OUT), jnp.float32),
        scratch_shapes=[pltpu.VMEM((N, H), jnp.float32)],
    )(p, xl, exs, b, gamma, beta, wfc, bfc, mask)


def kernel(x, edge_index, mask, W1, att_src1, att_dst1, b1, gamma1, beta1,
           W2, att_src2, att_dst2, b2, gamma2, beta2, Wfc, bfc):
    src = edge_index[0].reshape(NS, EPT)
    dst = edge_index[1].reshape(NS, EPT)
    zeros = jnp.zeros((CHUNK_W,), jnp.float32)

    xl1, asrc1, adst1, amax1, exs1 = _pre1(
        x, W1, att_src1.reshape(H, 1), att_dst1.reshape(H, 1))
    p1 = _sc_scatter(src, dst, asrc1.reshape(N), adst1.reshape(N),
                     jnp.broadcast_to(amax1.reshape(1), (LANES,)), zeros)
    xl2, asrc2, adst2, amax2, exs2 = _midpre(
        p1.reshape(N, N), xl1, exs1, b1.reshape(1, H),
        gamma1.reshape(1, H), beta1.reshape(1, H), W2,
        att_src2.reshape(H, 1), att_dst2.reshape(H, 1))
    p2 = _sc_scatter(src, dst, asrc2.reshape(N), adst2.reshape(N),
                     jnp.broadcast_to(amax2.reshape(1), (LANES,)), zeros)
    out = _midpost(p2.reshape(N, N), xl2, exs2, b2.reshape(1, H),
                   gamma2.reshape(1, H), beta2.reshape(1, H),
                   Wfc, bfc.reshape(1, D_OUT), mask.reshape(N, 1))
    return out.reshape(64, 40, 2)


# speedup vs baseline: 2.6389x; 1.0013x over previous
import jax
import jax.numpy as jnp
from jax import lax
from jax.experimental import pallas as pl
from jax.experimental.pallas import tpu as pltpu
from jax.experimental.pallas import tpu_sc as plsc

N = 2560
E = 81920
H = 256
D_IN = 128
D_OUT = 2

NC = 2
NS = 16
LANES = 16

NCHUNK = 8
ROWS = N // NCHUNK
CHUNK_W = ROWS * N
SLICE_W = CHUNK_W // NS
EPT = E // NS

def _dot(a, b):
    ah = a.astype(jnp.bfloat16)
    al = (a - ah.astype(jnp.float32)).astype(jnp.bfloat16)
    bh = b.astype(jnp.bfloat16)
    bl = (b - bh.astype(jnp.float32)).astype(jnp.bfloat16)
    dims = (((1,), (0,)), ((), ()))

    def d(u, v):
        return jax.lax.dot_general(u, v, dims,
                                   preferred_element_type=jnp.float32)

    return d(ah, bh) + (d(ah, bl) + d(al, bh))


def _dot1(a, b):
    return jax.lax.dot_general(a.astype(jnp.bfloat16), b.astype(jnp.bfloat16),
                               (((1,), (0,)), ((), ())),
                               preferred_element_type=jnp.float32)


def _lrelu(v):
    return jnp.maximum(v, 0.2 * v)


def _pre1_body(x_ref, w_ref, as_ref, ad_ref,
               xl_ref, asrc_ref, adst_ref, amax_ref, exs_ref):
    xl = _dot1(x_ref[...], w_ref[...])
    xl_ref[...] = xl
    a_s = _dot(xl, as_ref[...])
    a_d = _dot(xl, ad_ref[...])
    asrc_ref[...] = a_s
    adst_ref[...] = a_d
    amax = jnp.max(a_s)
    amax_ref[...] = jnp.full((1, 1), amax, jnp.float32)
    m = _lrelu(amax + a_d)
    exs_ref[...] = jnp.exp(_lrelu(a_s + a_d) - m)


def _pre1(x, w, att_s, att_d):
    return pl.pallas_call(
        _pre1_body,
        out_shape=[
            jax.ShapeDtypeStruct((N, H), jnp.float32),
            jax.ShapeDtypeStruct((N, 1), jnp.float32),
            jax.ShapeDtypeStruct((N, 1), jnp.float32),
            jax.ShapeDtypeStruct((1, 1), jnp.float32),
            jax.ShapeDtypeStruct((N, 1), jnp.float32),
        ],
    )(x, w, att_s, att_d)


def _sc_scatter_body(src_hbm, dst_hbm, asrc_hbm, adst_hbm, amax_hbm, zeros_hbm,
                     p_hbm,
                     amax_v, src_v, dst_v, ag_v, dg_v, ex_v, val_v, idx_v,
                     dmy_val, dmy_idx, p_sh, sem):
    c = lax.axis_index("c")
    s = lax.axis_index("s")
    pltpu.sync_copy(amax_hbm, amax_v)
    pltpu.sync_copy(src_hbm.at[s], src_v)
    pltpu.sync_copy(dst_hbm.at[s], dst_v)
    amx = amax_v[...]

    cp1 = pltpu.async_copy(asrc_hbm.at[src_v], ag_v, sem)
    cp2 = pltpu.async_copy(adst_hbm.at[dst_v], dg_v, sem)
    cp1.wait()
    cp2.wait()

    def ex_row(r, carry):
        sl = pl.ds(r * LANES, LANES)
        ag = ag_v[sl]
        dg = dg_v[sl]
        t = ag + dg
        alpha = jnp.maximum(t, 0.2 * t)
        m0 = amx + dg
        m = jnp.maximum(m0, 0.2 * m0)
        ex_v[sl] = jnp.exp(alpha - m)
        return carry

    lax.fori_loop(0, EPT // LANES, ex_row, 0)

    def zero_dummy(r, carry):
        sl = pl.ds(r * LANES, LANES)
        dmy_val[sl] = jnp.zeros((LANES,), jnp.float32)
        dmy_idx[sl] = jnp.zeros((LANES,), jnp.int32)
        return carry

    lax.fori_loop(0, 128 // LANES, zero_dummy, 0)

    for k in range(NCHUNK // NC):
        cid = c * (NCHUNK // NC) + k
        lo = cid * ROWS
        zcp = pltpu.async_copy(zeros_hbm.at[pl.ds(s * SLICE_W, SLICE_W)],
                               p_sh.at[pl.ds(s * SLICE_W, SLICE_W)], sem)
        kbase = k * EPT

        def compute_row(r, carry):
            sl = pl.ds(kbase + r * LANES, LANES)
            sle = pl.ds(r * LANES, LANES)
            sv = src_v[sle]
            dv = dst_v[sle]
            ex = ex_v[sle]
            dloc = dv - lo
            inrng = (dloc >= 0) & (dloc < ROWS)
            spread = s * EPT + r * LANES + lax.iota(jnp.int32, LANES)
            val_v[sl] = jnp.where(inrng, ex, 0.0)
            idx_v[sl] = jnp.where(inrng, dloc * N + sv, spread)
            return carry

        lax.fori_loop(0, EPT // LANES, compute_row, 0)
        zcp.wait()
        plsc.subcore_barrier()

        pltpu.sync_copy(val_v.at[pl.ds(kbase, EPT)],
                        p_sh.at[idx_v.at[pl.ds(kbase, EPT)]], add=True)
        pltpu.sync_copy(dmy_val, p_sh.at[dmy_idx], add=True)
        pltpu.sync_copy(dmy_val, p_sh.at[dmy_idx], add=True)
        plsc.subcore_barrier()
        pltpu.sync_copy(p_sh.at[pl.ds(s * SLICE_W, SLICE_W)],
                        p_hbm.at[pl.ds(cid * CHUNK_W + s * SLICE_W, SLICE_W)])
        if k < NCHUNK // NC - 1:
            plsc.subcore_barrier()


def _sc_scatter(src, dst, asrc, adst, amax16, zeros):
    mesh = plsc.VectorSubcoreMesh(core_axis_name="c", subcore_axis_name="s")
    f = pl.kernel(
        _sc_scatter_body,
        out_type=jax.ShapeDtypeStruct((N * N,), jnp.float32),
        mesh=mesh,
        scratch_types=[
            pltpu.VMEM((LANES,), jnp.float32),
            pltpu.VMEM((EPT,), jnp.int32),
            pltpu.VMEM((EPT,), jnp.int32),
            pltpu.VMEM((EPT,), jnp.float32),
            pltpu.VMEM((EPT,), jnp.float32),
            pltpu.VMEM((EPT,), jnp.float32),
            pltpu.VMEM(((NCHUNK // NC) * EPT,), jnp.float32),
            pltpu.VMEM(((NCHUNK // NC) * EPT,), jnp.int32),
            pltpu.VMEM((128,), jnp.float32),
            pltpu.VMEM((128,), jnp.int32),
            pltpu.VMEM_SHARED((CHUNK_W,), jnp.float32),
            pltpu.SemaphoreType.DMA,
        ],
    )
    return f(src, dst, asrc, adst, amax16, zeros)


_MID_BLK = 320
_MID_G = N // _MID_BLK


def _mid_block(p_ref, xl_ref, exs_ref, b_ref, raw_s):
    i = pl.program_id(0)
    p = p_ref[...]
    acc = _dot(p, xl_ref[...])
    rows = pl.ds(i * _MID_BLK, _MID_BLK)
    exs = exs_ref[rows, :]
    denom = jnp.sum(p, axis=1, keepdims=True) + exs + 1e-16
    raw_s[rows, :] = (acc + exs * xl_ref[rows, :]) / denom + b_ref[...]


def _bn_relu(r, g, bt):
    mean = jnp.mean(r, axis=0, keepdims=True)
    d = r - mean
    var = jnp.mean(d * d, axis=0, keepdims=True)
    return jnp.maximum(d * jax.lax.rsqrt(var + 1e-5) * g + bt, 0.0)


def _midpre_body(p_ref, xl_ref, exs_ref, b_ref, g_ref, bt_ref, w_ref,
                 as_ref, ad_ref,
                 xl2_ref, asrc_ref, adst_ref, amax_ref, exs2_ref, raw_s):
    _mid_block(p_ref, xl_ref, exs_ref, b_ref, raw_s)

    @pl.when(pl.program_id(0) == _MID_G - 1)
    def _():
        h = _bn_relu(raw_s[...], g_ref[...], bt_ref[...])
        xl = _dot1(h, w_ref[...])
        xl2_ref[...] = xl
        a_s = _dot(xl, as_ref[...])
        a_d = _dot(xl, ad_ref[...])
        asrc_ref[...] = a_s
        adst_ref[...] = a_d
        amax = jnp.max(a_s)
        amax_ref[...] = jnp.full((1, 1), amax, jnp.float32)
        m = _lrelu(amax + a_d)
        exs2_ref[...] = jnp.exp(_lrelu(a_s + a_d) - m)


def _midpre(p, xl, exs, b, gamma, beta, w, att_s, att_d):
    full = pl.BlockSpec(index_map=lambda i: (0, 0))
    return pl.pallas_call(
        _midpre_body,
        grid=(_MID_G,),
        in_specs=[
            pl.BlockSpec((_MID_BLK, N), lambda i: (i, 0)),
            pl.BlockSpec((N, H), lambda i: (0, 0)),
            pl.BlockSpec((N, 1), lambda i: (0, 0)),
            pl.BlockSpec((1, H), lambda i: (0, 0)),
            pl.BlockSpec((1, H), lambda i: (0, 0)),
            pl.BlockSpec((1, H), lambda i: (0, 0)),
            pl.BlockSpec((H, H), lambda i: (0, 0)),
            pl.BlockSpec((H, 1), lambda i: (0, 0)),
            pl.BlockSpec((H, 1), lambda i: (0, 0)),
        ],
        out_specs=[
            pl.BlockSpec((N, H), lambda i: (0, 0)),
            pl.BlockSpec((N, 1), lambda i: (0, 0)),
            pl.BlockSpec((N, 1), lambda i: (0, 0)),
            pl.BlockSpec((1, 1), lambda i: (0, 0)),
            pl.BlockSpec((N, 1), lambda i: (0, 0)),
        ],
        out_shape=[
            jax.ShapeDtypeStruct((N, H), jnp.float32),
            jax.ShapeDtypeStruct((N, 1), jnp.float32),
            jax.ShapeDtypeStruct((N, 1), jnp.float32),
            jax.ShapeDtypeStruct((1, 1), jnp.float32),
            jax.ShapeDtypeStruct((N, 1), jnp.float32),
        ],
        scratch_shapes=[pltpu.VMEM((N, H), jnp.float32)],
    )(p, xl, exs, b, gamma, beta, w, att_s, att_d)


def _midpost_body(p_ref, xl_ref, exs_ref, b_ref, g_ref, bt_ref, wfc_ref,
                  bfc_ref, mask_ref, out_ref, raw_s):
    _mid_block(p_ref, xl_ref, exs_ref, b_ref, raw_s)

    @pl.when(pl.program_id(0) == _MID_G - 1)
    def _():
        h = _bn_relu(raw_s[...], g_ref[...], bt_ref[...])
        o = _dot1(h, wfc_ref[...]) + bfc_ref[...]
        out_ref[...] = o * mask_ref[...]


def _midpost(p, xl, exs, b, gamma, beta, wfc, bfc, mask):
    return pl.pallas_call(
        _midpost_body,
        grid=(_MID_G,),
        in_specs=[
            pl.BlockSpec((_MID_BLK, N), lambda i: (i, 0)),
            pl.BlockSpec((N, H), lambda i: (0, 0)),
            pl.BlockSpec((N, 1), lambda i: (0, 0)),
            pl.BlockSpec((1, H), lambda i: (0, 0)),
            pl.BlockSpec((1, H), lambda i: (0, 0)),
            pl.BlockSpec((1, H), lambda i: (0, 0)),
            pl.BlockSpec((H, D_OUT), lambda i: (0, 0)),
            pl.BlockSpec((1, D_OUT), lambda i: (0, 0)),
            pl.BlockSpec((N, 1), lambda i: (0, 0)),
        ],
        out_specs=pl.BlockSpec((N, D_OUT), lambda i: (0, 0)),
        out_shape=jax.ShapeDtypeStruct((N, D_OUT), jnp.float32),
        scratch_shapes=[pltpu.VMEM((N, H), jnp.float32)],
    )(p, xl, exs, b, gamma, beta, wfc, bfc, mask)


def kernel(x, edge_index, mask, W1, att_src1, att_dst1, b1, gamma1, beta1,
           W2, att_src2, att_dst2, b2, gamma2, beta2, Wfc, bfc):
    src = edge_index[0].reshape(NS, EPT)
    dst = edge_index[1].reshape(NS, EPT)
    zeros = jnp.zeros((CHUNK_W,), jnp.float32)

    xl1, asrc1, adst1, amax1, exs1 = _pre1(
        x, W1, att_src1.reshape(H, 1), att_dst1.reshape(H, 1))
    p1 = _sc_scatter(src, dst, asrc1.reshape(N), adst1.reshape(N),
                     jnp.broadcast_to(amax1.reshape(1), (LANES,)), zeros)
    xl2, asrc2, adst2, amax2, exs2 = _midpre(
        p1.reshape(N, N), xl1, exs1, b1.reshape(1, H),
        gamma1.reshape(1, H), beta1.reshape(1, H), W2,
        att_src2.reshape(H, 1), att_dst2.reshape(H, 1))
    p2 = _sc_scatter(src, dst, asrc2.reshape(N), adst2.reshape(N),
                     jnp.broadcast_to(amax2.reshape(1), (LANES,)), zeros)
    out = _midpost(p2.reshape(N, N), xl2, exs2, b2.reshape(1, H),
                   gamma2.reshape(1, H), beta2.reshape(1, H),
                   Wfc, bfc.reshape(1, D_OUT), mask.reshape(N, 1))
    return out.reshape(64, 40, 2)
